# baseline (speedup 1.0000x reference)
import functools

import jax
import jax.numpy as jnp
from jax import lax
from jax.experimental import pallas as pl
from jax.experimental.pallas import tpu as pltpu
from jax.experimental.pallas import tpu_sc as plsc

N = 10000
E = 320000
D = 128
H = 128
B = 100
NPG = N // B
EPG = E // B
K = 3
C = 10
NP = 128
NW = 32


def _sc_hist_body(edge_ref, zero_ref, a_ref, src_v, dst_v, acc_v):
    c = lax.axis_index("c")
    s = lax.axis_index("s")
    wid = s * 2 + c

    def do_graph(r, carry):
        g = r * NW + wid

        @pl.when(g < B)
        def _():
            pltpu.sync_copy(zero_ref, acc_v)
            pltpu.sync_copy(edge_ref.at[0, pl.ds(g * EPG, EPG)], src_v)
            pltpu.sync_copy(edge_ref.at[1, pl.ds(g * EPG, EPG)], dst_v)
            base = g * NPG
            ones = jnp.ones((16,), jnp.float32)

            def scat(i, carry2):
                sl = src_v[pl.ds(i * 16, 16)] - base
                dl = dst_v[pl.ds(i * 16, 16)] - base
                plsc.addupdate_scatter(acc_v, [dl * NP + sl], ones)
                return carry2

            lax.fori_loop(0, EPG // 16, scat, 0)
            pltpu.sync_copy(acc_v, a_ref.at[g])

        return carry

    lax.fori_loop(0, (B + NW - 1) // NW, do_graph, 0)


def _build_adjacency(edge_index, zeros_tile):
    mesh = plsc.VectorSubcoreMesh(core_axis_name="c", subcore_axis_name="s")
    return pl.kernel(
        _sc_hist_body,
        out_type=jax.ShapeDtypeStruct((B, NP * NP), jnp.float32),
        mesh=mesh,
        compiler_params=pltpu.CompilerParams(needs_layout_passes=False),
        scratch_types=[
            pltpu.VMEM((EPG,), jnp.int32),
            pltpu.VMEM((EPG,), jnp.int32),
            pltpu.VMEM((NP * NP,), jnp.float32),
        ],
    )(edge_index, zeros_tile)


def _dot_x(a, b):
    return lax.dot(a, b, precision=lax.Precision.HIGHEST,
                   preferred_element_type=jnp.float32)


def _dot_d(a, b):
    return lax.dot(a, b, preferred_element_type=jnp.float32)


GPB = 4


def _dot_t(a, b):
    return lax.dot_general(a, b, (((0,), (0,)), ((), ())),
                           preferred_element_type=jnp.float32)


def _tc_gnn_body(a_ref, x_ref, w1_ref, b1_ref, w2_ref, b2_ref,
                 wcp_ref, bc_ref, wcls_ref, bcls_ref, out_ref):
    sub_i = lax.broadcasted_iota(jnp.int32, (NP, 1), 0)
    lane_i = lax.broadcasted_iota(jnp.int32, (1, NP), 1)
    valid = sub_i < NPG
    ones_col = jnp.ones((NP, 1), jnp.float32)
    ones_row = jnp.ones((1, NP), jnp.float32)
    eye = jnp.where(sub_i == lane_i, 1.0, 0.0)
    for gg in range(GPB):
        a = a_ref[gg]
        deg_in = _dot_d(a, ones_col)
        deg_out = _dot_d(ones_row, a)
        in_n = lax.rsqrt(jnp.maximum(deg_in, 1.0))
        out_n = lax.rsqrt(jnp.maximum(deg_out, 1.0))
        an = a * in_n * out_n

        h1 = jnp.maximum(_dot_d(_dot_x(an, x_ref[gg]), w1_ref[:]) + b1_ref[:], 0.0)
        h2 = jnp.maximum(_dot_d(_dot_x(an, h1), w2_ref[:]) + b2_ref[:], 0.0)

        key = jnp.where(valid, h2[:, H - 1:H], -1.0)
        rows = []
        for _ in range(K):
            m = jnp.max(key)
            idx = jnp.min(jnp.where(key == m, sub_i, NP))
            sel = sub_i == idx
            rows.append(jnp.sum(jnp.where(sel, h2, 0.0), axis=0, keepdims=True))
            key = jnp.where(sel, -1.0, key)
        hp = jnp.concatenate(rows, axis=1)

        hc = jnp.maximum(_dot_d(hp, wcp_ref[:]) + bc_ref[:], 0.0)
        out_ref[gg] = _dot_d(hc, wcls_ref[:]) + bcls_ref[:]


def _run_gnn(a, xp, w1, b1, w2, b2, wcp, bc, wcls, bcls):
    full = lambda shape: pl.BlockSpec(shape, lambda g: (0,) * len(shape))
    return pl.pallas_call(
        _tc_gnn_body,
        grid=(B // GPB,),
        in_specs=[
            pl.BlockSpec((GPB, NP, NP), lambda g: (g, 0, 0)),
            pl.BlockSpec((GPB, NP, D), lambda g: (g, 0, 0)),
            full((D, H)), full((1, H)),
            full((H, H)), full((1, H)),
            full((K * H, H)), full((1, H)),
            full((H, 128)), full((1, 128)),
        ],
        out_specs=pl.BlockSpec((GPB, 1, 128), lambda g: (g, 0, 0)),
        out_shape=jax.ShapeDtypeStruct((B, 1, 128), jnp.float32),
    )(a, xp, w1, b1, w2, b2, wcp, bc, wcls, bcls).reshape(B, 128)


def kernel(x, edge_index, W1, b1, W2, b2, Wc, bc, Wcls, bcls):
    xp = jnp.pad(x.reshape(B, NPG, D), ((0, 0), (0, NP - NPG), (0, 0)))
    zeros_tile = jnp.zeros((NP * NP,), jnp.float32)
    wcp = Wc.reshape(H, H * K).T
    wcls_p = jnp.pad(Wcls, ((0, 0), (0, 128 - C)))
    bcls_p = jnp.pad(bcls, (0, 128 - C)).reshape(1, 128)

    a = _build_adjacency(edge_index, zeros_tile).reshape(B, NP, NP)
    out = _run_gnn(a, xp, W1, b1.reshape(1, H), W2, b2.reshape(1, H),
                   wcp, bc.reshape(1, H), wcls_p, bcls_p)
    return out[:, :C]


# speedup vs baseline: 1.3038x; 1.3038x over previous
import functools

import jax
import jax.numpy as jnp
from jax import lax
from jax.experimental import pallas as pl
from jax.experimental.pallas import tpu as pltpu
from jax.experimental.pallas import tpu_sc as plsc

N = 10000
E = 320000
D = 128
H = 128
B = 100
NPG = N // B
EPG = E // B
K = 3
C = 10
NP = 128
NW = 32


def _sc_hist_body(edge_ref, zero_ref, a_ref, src_v, dst_v, acc_v):
    c = lax.axis_index("c")
    s = lax.axis_index("s")
    wid = s * 2 + c

    def do_graph(r, carry):
        g = r * NW + wid

        @pl.when(g < B)
        def _():
            pltpu.sync_copy(zero_ref, acc_v)
            pltpu.sync_copy(edge_ref.at[0, pl.ds(g * EPG, EPG)], src_v)
            pltpu.sync_copy(edge_ref.at[1, pl.ds(g * EPG, EPG)], dst_v)
            base = g * NPG
            ones = jnp.ones((16,), jnp.float32)

            def scat(i, carry2):
                sl = src_v[pl.ds(i * 16, 16)] - base
                dl = dst_v[pl.ds(i * 16, 16)] - base
                plsc.addupdate_scatter(acc_v, [dl * NP + sl], ones)
                return carry2

            lax.fori_loop(0, EPG // 16, scat, 0)
            pltpu.sync_copy(acc_v, a_ref.at[g])

        return carry

    lax.fori_loop(0, (B + NW - 1) // NW, do_graph, 0)


def _build_adjacency(edge_index, zeros_tile):
    mesh = plsc.VectorSubcoreMesh(core_axis_name="c", subcore_axis_name="s")
    return pl.kernel(
        _sc_hist_body,
        out_type=jax.ShapeDtypeStruct((B, NP * NP), jnp.float32),
        mesh=mesh,
        compiler_params=pltpu.CompilerParams(needs_layout_passes=False),
        scratch_types=[
            pltpu.VMEM((EPG,), jnp.int32),
            pltpu.VMEM((EPG,), jnp.int32),
            pltpu.VMEM((NP * NP,), jnp.float32),
        ],
    )(edge_index, zeros_tile)


def _dot_x(a, b):
    return lax.dot(a, b, precision=lax.Precision.HIGHEST,
                   preferred_element_type=jnp.float32)


def _dot_d(a, b):
    return lax.dot(a, b, preferred_element_type=jnp.float32)


GPB = 4


def _dot_t(a, b):
    return lax.dot_general(a, b, (((0,), (0,)), ((), ())),
                           preferred_element_type=jnp.float32)


def _tc_gnn_body(a_ref, x_ref, w1_ref, b1_ref, w2_ref, b2_ref,
                 wcp_ref, bc_ref, wcls_ref, bcls_ref, out_ref):
    sub_i = lax.broadcasted_iota(jnp.int32, (NP, 1), 0)
    lane_i = lax.broadcasted_iota(jnp.int32, (1, NP), 1)
    valid = sub_i < NPG
    ones_col = jnp.ones((NP, 1), jnp.float32)
    ones_row = jnp.ones((1, NP), jnp.float32)
    eye = jnp.where(sub_i == lane_i, 1.0, 0.0)
    for gg in range(GPB):
        a = a_ref[gg]
        deg_in = _dot_d(a, ones_col)
        deg_out = _dot_d(ones_row, a)
        in_n = lax.rsqrt(jnp.maximum(deg_in, 1.0))
        out_n = lax.rsqrt(jnp.maximum(deg_out, 1.0))
        an = a * in_n * out_n

        h1 = jnp.maximum(_dot_d(_dot_x(an, x_ref[gg]), w1_ref[:]) + b1_ref[:], 0.0)
        h2 = jnp.maximum(_dot_d(_dot_x(an, h1), w2_ref[:]) + b2_ref[:], 0.0)

        key = jnp.where(valid, h2[:, H - 1:H], -1.0)
        key_row = lax.transpose(key, (1, 0))
        cmp = jnp.where((key > key_row)
                        | ((key == key_row) & (sub_i < lane_i)), 1.0, 0.0)
        rank_row = _dot_d(ones_row, cmp)
        perm = jnp.where(rank_row == sub_i.astype(jnp.float32),
                         1.0, 0.0)
        top = _dot_d(perm, h2)
        hp = jnp.concatenate([top[0:1, :], top[1:2, :], top[2:3, :]], axis=1)

        hc = jnp.maximum(_dot_d(hp, wcp_ref[:]) + bc_ref[:], 0.0)
        out_ref[gg] = _dot_d(hc, wcls_ref[:]) + bcls_ref[:]


def _run_gnn(a, xp, w1, b1, w2, b2, wcp, bc, wcls, bcls):
    full = lambda shape: pl.BlockSpec(shape, lambda g: (0,) * len(shape))
    return pl.pallas_call(
        _tc_gnn_body,
        grid=(B // GPB,),
        in_specs=[
            pl.BlockSpec((GPB, NP, NP), lambda g: (g, 0, 0)),
            pl.BlockSpec((GPB, NP, D), lambda g: (g, 0, 0)),
            full((D, H)), full((1, H)),
            full((H, H)), full((1, H)),
            full((K * H, H)), full((1, H)),
            full((H, 128)), full((1, 128)),
        ],
        out_specs=pl.BlockSpec((GPB, 1, 128), lambda g: (g, 0, 0)),
        out_shape=jax.ShapeDtypeStruct((B, 1, 128), jnp.float32),
    )(a, xp, w1, b1, w2, b2, wcp, bc, wcls, bcls).reshape(B, 128)


def kernel(x, edge_index, W1, b1, W2, b2, Wc, bc, Wcls, bcls):
    xp = jnp.pad(x.reshape(B, NPG, D), ((0, 0), (0, NP - NPG), (0, 0)))
    zeros_tile = jnp.zeros((NP * NP,), jnp.float32)
    wcp = Wc.reshape(H, H * K).T
    wcls_p = jnp.pad(Wcls, ((0, 0), (0, 128 - C)))
    bcls_p = jnp.pad(bcls, (0, 128 - C)).reshape(1, 128)

    a = _build_adjacency(edge_index, zeros_tile).reshape(B, NP, NP)
    out = _run_gnn(a, xp, W1, b1.reshape(1, H), W2, b2.reshape(1, H),
                   wcp, bc.reshape(1, H), wcls_p, bcls_p)
    return out[:, :C]


# speedup vs baseline: 1.4327x; 1.0989x over previous
import functools

import jax
import jax.numpy as jnp
from jax import lax
from jax.experimental import pallas as pl
from jax.experimental.pallas import tpu as pltpu
from jax.experimental.pallas import tpu_sc as plsc

N = 10000
E = 320000
D = 128
H = 128
B = 100
NPG = N // B
EPG = E // B
K = 3
C = 10
NP = 128
NW = 32


def _sc_hist_body(edge_ref, zero_ref, a_ref, src_v, dst_v, acc_v):
    c = lax.axis_index("c")
    s = lax.axis_index("s")
    wid = s * 2 + c

    def do_graph(r, carry):
        g = r * NW + wid

        @pl.when(g < B)
        def _():
            pltpu.sync_copy(zero_ref, acc_v)
            pltpu.sync_copy(edge_ref.at[0, pl.ds(g * EPG, EPG)], src_v)
            pltpu.sync_copy(edge_ref.at[1, pl.ds(g * EPG, EPG)], dst_v)
            base = g * NPG
            ones = jnp.ones((16,), jnp.float32)

            def scat(i, carry2):
                sl = src_v[pl.ds(i * 16, 16)] - base
                dl = dst_v[pl.ds(i * 16, 16)] - base
                plsc.addupdate_scatter(acc_v, [dl, sl], ones)
                return carry2

            lax.fori_loop(0, EPG // 16, scat, 0)
            pltpu.sync_copy(acc_v, a_ref.at[g])

        return carry

    lax.fori_loop(0, (B + NW - 1) // NW, do_graph, 0)


def _build_adjacency(edge_index, zeros_tile):
    mesh = plsc.VectorSubcoreMesh(core_axis_name="c", subcore_axis_name="s")
    return pl.kernel(
        _sc_hist_body,
        out_type=jax.ShapeDtypeStruct((B, NP, NP), jnp.float32),
        mesh=mesh,
        compiler_params=pltpu.CompilerParams(needs_layout_passes=False),
        scratch_types=[
            pltpu.VMEM((EPG,), jnp.int32),
            pltpu.VMEM((EPG,), jnp.int32),
            pltpu.VMEM((NP, NP), jnp.float32),
        ],
    )(edge_index, zeros_tile)


def _dot_x(a, b):
    return lax.dot(a, b, precision=lax.Precision.HIGHEST,
                   preferred_element_type=jnp.float32)


def _dot_d(a, b):
    return lax.dot(a, b, preferred_element_type=jnp.float32)


GPB = 4


def _dot_t(a, b):
    return lax.dot_general(a, b, (((0,), (0,)), ((), ())),
                           preferred_element_type=jnp.float32)


def _tc_gnn_body(a_ref, x_ref, w1_ref, b1_ref, w2_ref, b2_ref,
                 wcp_ref, bc_ref, wcls_ref, bcls_ref, out_ref):
    sub_i = lax.broadcasted_iota(jnp.int32, (NP, 1), 0)
    lane_i = lax.broadcasted_iota(jnp.int32, (1, NP), 1)
    valid = sub_i < NPG
    ones_col = jnp.ones((NP, 1), jnp.float32)
    ones_row = jnp.ones((1, NP), jnp.float32)
    eye = jnp.where(sub_i == lane_i, 1.0, 0.0)
    for gg in range(GPB):
        a = a_ref[gg]
        deg_in = _dot_d(a, ones_col)
        deg_out = _dot_d(ones_row, a)
        in_n = lax.rsqrt(jnp.maximum(deg_in, 1.0))
        out_n = lax.rsqrt(jnp.maximum(deg_out, 1.0))
        an = a * in_n * out_n

        h1 = jnp.maximum(_dot_d(_dot_x(an, x_ref[gg]), w1_ref[:]) + b1_ref[:], 0.0)
        h2 = jnp.maximum(_dot_d(_dot_x(an, h1), w2_ref[:]) + b2_ref[:], 0.0)

        key = jnp.where(valid, h2[:, H - 1:H], -1.0)
        key_row = lax.transpose(key, (1, 0))
        cmp = jnp.where((key > key_row)
                        | ((key == key_row) & (sub_i < lane_i)), 1.0, 0.0)
        rank_row = _dot_d(ones_row, cmp)
        perm = jnp.where(rank_row == sub_i.astype(jnp.float32),
                         1.0, 0.0)
        top = _dot_d(perm, h2)
        hp = jnp.concatenate([top[0:1, :], top[1:2, :], top[2:3, :]], axis=1)

        hc = jnp.maximum(_dot_d(hp, wcp_ref[:]) + bc_ref[:], 0.0)
        out_ref[gg] = _dot_d(hc, wcls_ref[:]) + bcls_ref[:]


def _run_gnn(a, xp, w1, b1, w2, b2, wcp, bc, wcls, bcls):
    full = lambda shape: pl.BlockSpec(shape, lambda g: (0,) * len(shape))
    return pl.pallas_call(
        _tc_gnn_body,
        grid=(B // GPB,),
        in_specs=[
            pl.BlockSpec((GPB, NP, NP), lambda g: (g, 0, 0)),
            pl.BlockSpec((GPB, NP, D), lambda g: (g, 0, 0)),
            full((D, H)), full((1, H)),
            full((H, H)), full((1, H)),
            full((K * H, H)), full((1, H)),
            full((H, 128)), full((1, 128)),
        ],
        out_specs=pl.BlockSpec((GPB, 1, 128), lambda g: (g, 0, 0)),
        out_shape=jax.ShapeDtypeStruct((B, 1, 128), jnp.float32),
    )(a, xp, w1, b1, w2, b2, wcp, bc, wcls, bcls).reshape(B, 128)


def kernel(x, edge_index, W1, b1, W2, b2, Wc, bc, Wcls, bcls):
    xp = jnp.pad(x.reshape(B, NPG, D), ((0, 0), (0, NP - NPG), (0, 0)))
    zeros_tile = jnp.zeros((NP, NP), jnp.float32)
    wcp = Wc.reshape(H, H * K).T
    wcls_p = jnp.pad(Wcls, ((0, 0), (0, 128 - C)))
    bcls_p = jnp.pad(bcls, (0, 128 - C)).reshape(1, 128)

    a = _build_adjacency(edge_index, zeros_tile)
    out = _run_gnn(a, xp, W1, b1.reshape(1, H), W2, b2.reshape(1, H),
                   wcp, bc.reshape(1, H), wcls_p, bcls_p)
    return out[:, :C]


# speedup vs baseline: 1.4482x; 1.0108x over previous
import functools

import jax
import jax.numpy as jnp
from jax import lax
from jax.experimental import pallas as pl
from jax.experimental.pallas import tpu as pltpu
from jax.experimental.pallas import tpu_sc as plsc

N = 10000
E = 320000
D = 128
H = 128
B = 100
NPG = N // B
EPG = E // B
K = 3
C = 10
NP = 128
NW = 32


def _sc_hist_body(edge_ref, zero_ref, a_ref, src_v, dst_v, acc_v):
    c = lax.axis_index("c")
    s = lax.axis_index("s")
    wid = s * 2 + c

    def do_graph(r, carry):
        g = r * NW + wid

        @pl.when(g < B)
        def _():
            pltpu.sync_copy(zero_ref, acc_v)
            pltpu.sync_copy(edge_ref.at[0, pl.ds(g * EPG, EPG)], src_v)
            pltpu.sync_copy(edge_ref.at[1, pl.ds(g * EPG, EPG)], dst_v)
            base = g * NPG
            ones = jnp.ones((16,), jnp.float32)

            def scat(i, carry2):
                sl = src_v[pl.ds(i * 16, 16)] - base
                dl = dst_v[pl.ds(i * 16, 16)] - base
                plsc.addupdate_scatter(acc_v, [dl, sl], ones)
                return carry2

            lax.fori_loop(0, EPG // 16, scat, 0)
            pltpu.sync_copy(acc_v, a_ref.at[g])

        return carry

    lax.fori_loop(0, (B + NW - 1) // NW, do_graph, 0)


def _build_adjacency(edge_index, zeros_tile):
    mesh = plsc.VectorSubcoreMesh(core_axis_name="c", subcore_axis_name="s")
    return pl.kernel(
        _sc_hist_body,
        out_type=jax.ShapeDtypeStruct((B, NP, NP), jnp.float32),
        mesh=mesh,
        compiler_params=pltpu.CompilerParams(needs_layout_passes=False),
        scratch_types=[
            pltpu.VMEM((EPG,), jnp.int32),
            pltpu.VMEM((EPG,), jnp.int32),
            pltpu.VMEM((NP, NP), jnp.float32),
        ],
    )(edge_index, zeros_tile)


def _dot_x(a, b):
    return lax.dot(a, b, precision=lax.Precision.HIGHEST,
                   preferred_element_type=jnp.float32)


def _dot_d(a, b):
    return lax.dot(a, b, preferred_element_type=jnp.float32)


GPB = 4


def _dot_t(a, b):
    return lax.dot_general(a, b, (((0,), (0,)), ((), ())),
                           preferred_element_type=jnp.float32)


def _tc_gnn_body(a_ref, x_ref, w1_ref, b1_ref, w2_ref, b2_ref,
                 wcp_ref, bc_ref, wcls_ref, bcls_ref, out_ref):
    sub_i = lax.broadcasted_iota(jnp.int32, (NP, 1), 0)
    lane_i = lax.broadcasted_iota(jnp.int32, (1, NP), 1)
    valid = sub_i < NPG
    ones_col = jnp.ones((NP, 1), jnp.float32)
    ones_row = jnp.ones((1, NP), jnp.float32)
    eye = jnp.where(sub_i == lane_i, 1.0, 0.0)
    for gg in range(GPB):
        a = a_ref[gg]
        deg_in = _dot_d(a, ones_col)
        deg_out = _dot_d(ones_row, a)
        in_n = lax.rsqrt(jnp.maximum(deg_in, 1.0))
        out_n = lax.rsqrt(jnp.maximum(deg_out, 1.0))
        an = a * in_n * out_n

        h1 = jnp.maximum(_dot_d(_dot_x(an, x_ref[gg]), w1_ref[:]) + b1_ref[:], 0.0)
        h2 = jnp.maximum(_dot_d(_dot_x(an, h1), w2_ref[:]) + b2_ref[:], 0.0)

        key = jnp.where(valid, h2[:, H - 1:H], -1.0)
        h2t = lax.transpose(h2, (1, 0))
        key_row = jnp.where(lane_i < NPG, h2t[H - 1:H, :], -1.0)
        cmp = jnp.where((key > key_row)
                        | ((key == key_row) & (sub_i < lane_i)), 1.0, 0.0)
        rank_row = _dot_d(ones_row, cmp)
        perm = jnp.where(rank_row == sub_i.astype(jnp.float32),
                         1.0, 0.0)
        top = _dot_d(perm, h2)
        hp = jnp.concatenate([top[0:1, :], top[1:2, :], top[2:3, :]], axis=1)

        hc = jnp.maximum(_dot_d(hp, wcp_ref[:]) + bc_ref[:], 0.0)
        out_ref[gg] = _dot_d(hc, wcls_ref[:]) + bcls_ref[:]


def _run_gnn(a, xp, w1, b1, w2, b2, wcp, bc, wcls, bcls):
    full = lambda shape: pl.BlockSpec(shape, lambda g: (0,) * len(shape))
    return pl.pallas_call(
        _tc_gnn_body,
        grid=(B // GPB,),
        in_specs=[
            pl.BlockSpec((GPB, NP, NP), lambda g: (g, 0, 0)),
            pl.BlockSpec((GPB, NP, D), lambda g: (g, 0, 0)),
            full((D, H)), full((1, H)),
            full((H, H)), full((1, H)),
            full((K * H, H)), full((1, H)),
            full((H, 128)), full((1, 128)),
        ],
        out_specs=pl.BlockSpec((GPB, 1, 128), lambda g: (g, 0, 0)),
        out_shape=jax.ShapeDtypeStruct((B, 1, 128), jnp.float32),
    )(a, xp, w1, b1, w2, b2, wcp, bc, wcls, bcls).reshape(B, 128)


def kernel(x, edge_index, W1, b1, W2, b2, Wc, bc, Wcls, bcls):
    xp = jnp.pad(x.reshape(B, NPG, D), ((0, 0), (0, NP - NPG), (0, 0)))
    zeros_tile = jnp.zeros((NP, NP), jnp.float32)
    wcp = Wc.reshape(H, H * K).T
    wcls_p = jnp.pad(Wcls, ((0, 0), (0, 128 - C)))
    bcls_p = jnp.pad(bcls, (0, 128 - C)).reshape(1, 128)

    a = _build_adjacency(edge_index, zeros_tile)
    out = _run_gnn(a, xp, W1, b1.reshape(1, H), W2, b2.reshape(1, H),
                   wcp, bc.reshape(1, H), wcls_p, bcls_p)
    return out[:, :C]


# speedup vs baseline: 1.4810x; 1.0227x over previous
import functools

import jax
import jax.numpy as jnp
from jax import lax
from jax.experimental import pallas as pl
from jax.experimental.pallas import tpu as pltpu
from jax.experimental.pallas import tpu_sc as plsc

N = 10000
E = 320000
D = 128
H = 128
B = 100
NPG = N // B
EPG = E // B
K = 3
C = 10
NP = 128
NW = 32


def _sc_hist_body(edge_ref, zero_ref, a_ref, src_v, dst_v, acc_v):
    c = lax.axis_index("c")
    s = lax.axis_index("s")
    wid = s * 2 + c

    def do_graph(r, carry):
        g = r * NW + wid

        @pl.when(g < B)
        def _():
            pltpu.sync_copy(zero_ref, acc_v)
            pltpu.sync_copy(edge_ref.at[0, pl.ds(g * EPG, EPG)], src_v)
            pltpu.sync_copy(edge_ref.at[1, pl.ds(g * EPG, EPG)], dst_v)
            base = g * NPG
            ones = jnp.ones((16,), jnp.float32)

            def scat(i, carry2):
                sl = src_v[pl.ds(i * 16, 16)] - base
                dl = dst_v[pl.ds(i * 16, 16)] - base
                plsc.addupdate_scatter(acc_v, [dl, sl], ones)
                return carry2

            lax.fori_loop(0, EPG // 16, scat, 0)
            pltpu.sync_copy(acc_v, a_ref.at[g])

        return carry

    lax.fori_loop(0, (B + NW - 1) // NW, do_graph, 0)


def _build_adjacency(edge_index, zeros_tile):
    mesh = plsc.VectorSubcoreMesh(core_axis_name="c", subcore_axis_name="s")
    return pl.kernel(
        _sc_hist_body,
        out_type=jax.ShapeDtypeStruct((B, NP, NP), jnp.float32),
        mesh=mesh,
        compiler_params=pltpu.CompilerParams(needs_layout_passes=False),
        scratch_types=[
            pltpu.VMEM((EPG,), jnp.int32),
            pltpu.VMEM((EPG,), jnp.int32),
            pltpu.VMEM((NP, NP), jnp.float32),
        ],
    )(edge_index, zeros_tile)


def _dot_x(a, b):
    return lax.dot(a, b, precision=lax.Precision.HIGHEST,
                   preferred_element_type=jnp.float32)


def _dot_d(a, b):
    return lax.dot(a, b, preferred_element_type=jnp.float32)


GPB = 10


def _dot_t(a, b):
    return lax.dot_general(a, b, (((0,), (0,)), ((), ())),
                           preferred_element_type=jnp.float32)


def _tc_gnn_body(a_ref, x_ref, w1_ref, b1_ref, w2_ref, b2_ref,
                 wcp_ref, bc_ref, wcls_ref, bcls_ref, out_ref):
    sub_i = lax.broadcasted_iota(jnp.int32, (NP, 1), 0)
    lane_i = lax.broadcasted_iota(jnp.int32, (1, NP), 1)
    valid = sub_i < NPG
    ones_col = jnp.ones((NP, 1), jnp.float32)
    ones_row = jnp.ones((1, NP), jnp.float32)
    eye = jnp.where(sub_i == lane_i, 1.0, 0.0)
    for gg in range(GPB):
        a = a_ref[gg]
        deg_in = _dot_d(a, ones_col)
        deg_out = _dot_d(ones_row, a)
        in_n = lax.rsqrt(jnp.maximum(deg_in, 1.0))
        out_n = lax.rsqrt(jnp.maximum(deg_out, 1.0))
        an = a * in_n * out_n

        h1 = jnp.maximum(_dot_d(_dot_x(an, x_ref[gg]), w1_ref[:]) + b1_ref[:], 0.0)
        h2 = jnp.maximum(_dot_d(_dot_x(an, h1), w2_ref[:]) + b2_ref[:], 0.0)

        key = jnp.where(valid, h2[:, H - 1:H], -1.0)
        h2t = lax.transpose(h2, (1, 0))
        key_row = jnp.where(lane_i < NPG, h2t[H - 1:H, :], -1.0)
        cmp = jnp.where((key > key_row)
                        | ((key == key_row) & (sub_i < lane_i)), 1.0, 0.0)
        rank_row = _dot_d(ones_row, cmp)
        perm = jnp.where(rank_row == sub_i.astype(jnp.float32),
                         1.0, 0.0)
        top = _dot_d(perm, h2)
        hp = jnp.concatenate([top[0:1, :], top[1:2, :], top[2:3, :]], axis=1)

        hc = jnp.maximum(_dot_d(hp, wcp_ref[:]) + bc_ref[:], 0.0)
        out_ref[gg] = _dot_d(hc, wcls_ref[:]) + bcls_ref[:]


def _run_gnn(a, xp, w1, b1, w2, b2, wcp, bc, wcls, bcls):
    full = lambda shape: pl.BlockSpec(shape, lambda g: (0,) * len(shape))
    return pl.pallas_call(
        _tc_gnn_body,
        grid=(B // GPB,),
        in_specs=[
            pl.BlockSpec((GPB, NP, NP), lambda g: (g, 0, 0)),
            pl.BlockSpec((GPB, NP, D), lambda g: (g, 0, 0)),
            full((D, H)), full((1, H)),
            full((H, H)), full((1, H)),
            full((K * H, H)), full((1, H)),
            full((H, 128)), full((1, 128)),
        ],
        out_specs=pl.BlockSpec((GPB, 1, 128), lambda g: (g, 0, 0)),
        out_shape=jax.ShapeDtypeStruct((B, 1, 128), jnp.float32),
    )(a, xp, w1, b1, w2, b2, wcp, bc, wcls, bcls).reshape(B, 128)


def kernel(x, edge_index, W1, b1, W2, b2, Wc, bc, Wcls, bcls):
    xp = jnp.pad(x.reshape(B, NPG, D), ((0, 0), (0, NP - NPG), (0, 0)))
    zeros_tile = jnp.zeros((NP, NP), jnp.float32)
    wcp = Wc.reshape(H, H * K).T
    wcls_p = jnp.pad(Wcls, ((0, 0), (0, 128 - C)))
    bcls_p = jnp.pad(bcls, (0, 128 - C)).reshape(1, 128)

    a = _build_adjacency(edge_index, zeros_tile)
    out = _run_gnn(a, xp, W1, b1.reshape(1, H), W2, b2.reshape(1, H),
                   wcp, bc.reshape(1, H), wcls_p, bcls_p)
    return out[:, :C]


# speedup vs baseline: 1.4821x; 1.0008x over previous
import jax
import jax.numpy as jnp
from jax import lax
from jax.experimental import pallas as pl
from jax.experimental.pallas import tpu as pltpu
from jax.experimental.pallas import tpu_sc as plsc

N = 10000
E = 320000
D = 128
H = 128
B = 100
NPG = N // B
EPG = E // B
K = 3
C = 10
NP = 128
NW = 32


def _sc_hist_body(edge_ref, zero_ref, a_ref, src_v, dst_v, acc_v):
    c = lax.axis_index("c")
    s = lax.axis_index("s")
    wid = s * 2 + c

    def do_graph(r, carry):
        g = r * NW + wid

        @pl.when(g < B)
        def _():
            pltpu.sync_copy(zero_ref, acc_v)
            pltpu.sync_copy(edge_ref.at[0, pl.ds(g * EPG, EPG)], src_v)
            pltpu.sync_copy(edge_ref.at[1, pl.ds(g * EPG, EPG)], dst_v)
            base = g * NPG
            ones = jnp.ones((16,), jnp.float32)

            def scat(i, carry2):
                sl = src_v[pl.ds(i * 16, 16)] - base
                dl = dst_v[pl.ds(i * 16, 16)] - base
                plsc.addupdate_scatter(acc_v, [dl, sl], ones)
                return carry2

            lax.fori_loop(0, EPG // 16, scat, 0)
            pltpu.sync_copy(acc_v, a_ref.at[g])

        return carry

    lax.fori_loop(0, (B + NW - 1) // NW, do_graph, 0)


def _build_adjacency(edge_index, zeros_tile):
    mesh = plsc.VectorSubcoreMesh(core_axis_name="c", subcore_axis_name="s")
    return pl.kernel(
        _sc_hist_body,
        out_type=jax.ShapeDtypeStruct((B, NP, NP), jnp.float32),
        mesh=mesh,
        compiler_params=pltpu.CompilerParams(needs_layout_passes=False),
        scratch_types=[
            pltpu.VMEM((EPG,), jnp.int32),
            pltpu.VMEM((EPG,), jnp.int32),
            pltpu.VMEM((NP, NP), jnp.float32),
        ],
    )(edge_index, zeros_tile)


def _dot_x(a, b):
    return lax.dot(a, b, precision=lax.Precision.HIGHEST,
                   preferred_element_type=jnp.float32)


def _dot_d(a, b):
    return lax.dot(a, b, preferred_element_type=jnp.float32)


GPB = 10


def _tc_gnn_body(a_ref, x_ref, w1_ref, b1_ref, w2_ref, b2_ref,
                 wcp_ref, bc_ref, wcls_ref, bcls_ref, out_ref):
    sub_i = lax.broadcasted_iota(jnp.int32, (NP, 1), 0)
    lane_i = lax.broadcasted_iota(jnp.int32, (1, NP), 1)
    valid = sub_i < NPG
    ones_col = jnp.ones((NP, 1), jnp.float32)
    ones_row = jnp.ones((1, NP), jnp.float32)
    for gg in range(GPB):
        a = a_ref[gg]
        deg_in = _dot_d(a, ones_col)
        deg_out = _dot_d(ones_row, a)
        in_n = lax.rsqrt(jnp.maximum(deg_in, 1.0))
        out_n = lax.rsqrt(jnp.maximum(deg_out, 1.0))
        an = a * in_n * out_n

        h1 = jnp.maximum(_dot_d(_dot_x(an, x_ref[gg]), w1_ref[:]) + b1_ref[:], 0.0)
        h2 = jnp.maximum(_dot_d(_dot_x(an, h1), w2_ref[:]) + b2_ref[:], 0.0)

        key = jnp.where(valid, h2[:, H - 1:H], -1.0)
        h2t = lax.transpose(h2, (1, 0))
        key_row = jnp.where(lane_i < NPG, h2t[H - 1:H, :], -1.0)
        cmp = jnp.where((key > key_row)
                        | ((key == key_row) & (sub_i < lane_i)), 1.0, 0.0)
        rank_row = _dot_d(ones_row, cmp)
        perm = jnp.where(rank_row == sub_i.astype(jnp.float32),
                         1.0, 0.0)
        top = _dot_d(perm, h2)
        hp = jnp.concatenate([top[0:1, :], top[1:2, :], top[2:3, :]], axis=1)

        hc = jnp.maximum(_dot_d(hp, wcp_ref[:]) + bc_ref[:], 0.0)
        out_ref[gg] = _dot_d(hc, wcls_ref[:]) + bcls_ref[:]


def _run_gnn(a, xp, w1, b1, w2, b2, wcp, bc, wcls, bcls):
    full = lambda shape: pl.BlockSpec(shape, lambda g: (0,) * len(shape))
    return pl.pallas_call(
        _tc_gnn_body,
        grid=(B // GPB,),
        in_specs=[
            pl.BlockSpec((GPB, NP, NP), lambda g: (g, 0, 0)),
            pl.BlockSpec((GPB, NP, D), lambda g: (g, 0, 0)),
            full((D, H)), full((1, H)),
            full((H, H)), full((1, H)),
            full((K * H, H)), full((1, H)),
            full((H, 128)), full((1, 128)),
        ],
        out_specs=pl.BlockSpec((GPB, 1, 128), lambda g: (g, 0, 0)),
        out_shape=jax.ShapeDtypeStruct((B, 1, 128), jnp.float32),
    )(a, xp, w1, b1, w2, b2, wcp, bc, wcls, bcls).reshape(B, 128)


def kernel(x, edge_index, W1, b1, W2, b2, Wc, bc, Wcls, bcls):
    xp = jnp.pad(x.reshape(B, NPG, D), ((0, 0), (0, NP - NPG), (0, 0)))
    zeros_tile = jnp.zeros((NP, NP), jnp.float32)
    wcp = Wc.reshape(H, H * K).T
    wcls_p = jnp.pad(Wcls, ((0, 0), (0, 128 - C)))
    bcls_p = jnp.pad(bcls, (0, 128 - C)).reshape(1, 128)

    a = _build_adjacency(edge_index, zeros_tile)
    out = _run_gnn(a, xp, W1, b1.reshape(1, H), W2, b2.reshape(1, H),
                   wcp, bc.reshape(1, H), wcls_p, bcls_p)
    return out[:, :C]
